# baseline (speedup 1.0000x reference)
import functools

import jax
import jax.numpy as jnp
from jax import lax
from jax.experimental import pallas as pl
from jax.experimental.pallas import tpu as pltpu
from jax.experimental.pallas import tpu_sc as plsc

C2 = 256
HW = 4096
QB = 512
NQ = HW // QB
PAD_ROWS = 8

NC = 2
NS = 16
NW = NC * NS
BPW = HW // NW


def _argmax_body(ltq_ref, ltf_ref, flagr_ref, flagc_ref, idx_ref, lnorm_ref):
    i = pl.program_id(0)

    @pl.when(i == 0)
    def _():
        ltf = ltf_ref[...]
        n2 = jnp.sum(ltf * ltf, axis=1, keepdims=True)
        norm = jnp.sqrt(n2) + 1e-8
        lnorm_ref[...] = ltf / norm

    cos = lax.dot_general(
        ltq_ref[...], lnorm_ref[...],
        (((1,), (1,)), ((), ())),
        preferred_element_type=jnp.float32,
    )
    cosm = jnp.where(flagr_ref[...] != 0, -jnp.inf, cos)
    m = jnp.max(cosm, axis=1, keepdims=True)
    qio = lax.broadcasted_iota(jnp.int32, (QB, HW), 1)
    idx = jnp.min(jnp.where(cosm == m, qio, jnp.int32(HW)), axis=1, keepdims=True)
    idx = jnp.where(flagc_ref[...] != 0, idx, jnp.int32(HW))
    idx_ref[...] = idx.reshape(1, QB, 1)


def _best_source_idx(lt, flag_row, flag_col):
    idx3 = pl.pallas_call(
        _argmax_body,
        grid=(NQ,),
        in_specs=[
            pl.BlockSpec((QB, C2), lambda i: (i, 0)),
            pl.BlockSpec((HW, C2), lambda i: (0, 0)),
            pl.BlockSpec((1, HW), lambda i: (0, 0)),
            pl.BlockSpec((QB, 1), lambda i: (i, 0)),
        ],
        out_specs=pl.BlockSpec((1, QB, 1), lambda i: (i, 0, 0)),
        out_shape=jax.ShapeDtypeStruct((NQ, QB, 1), jnp.int32),
        scratch_shapes=[pltpu.VMEM((HW, C2), jnp.float32)],
    )(lt, lt, flag_row, flag_col)
    return idx3.reshape(HW)


def _sc_gather_body(table_hbm, idx_hbm, out_hbm, idx_v, rows_v, sem):
    wid = lax.axis_index("s") * NC + lax.axis_index("c")
    base = wid * BPW
    pltpu.sync_copy(idx_hbm.at[pl.ds(base, BPW)], idx_v)
    pltpu.async_copy(table_hbm.at[idx_v], rows_v, sem).wait()
    pltpu.sync_copy(rows_v, out_hbm.at[pl.ds(base, BPW)])


@functools.cache
def _make_sc_gather():
    return pl.kernel(
        _sc_gather_body,
        out_type=jax.ShapeDtypeStruct((HW, C2), jnp.float32),
        mesh=plsc.VectorSubcoreMesh(core_axis_name="c", subcore_axis_name="s"),
        scratch_types=[
            pltpu.VMEM((BPW,), jnp.int32),
            pltpu.VMEM((BPW, C2), jnp.float32),
            pltpu.SemaphoreType.DMA,
        ],
    )


def kernel(input, mask):
    b, c, h, w = input.shape
    x2 = input.reshape(c, HW)
    former = x2[:C2]
    latter = x2[C2:]
    flag = (mask.reshape(HW) > 0).astype(jnp.float32)

    lt = latter.T
    idx = _best_source_idx(lt, flag.reshape(1, HW), flag.reshape(HW, 1))

    table = jnp.concatenate(
        [former.T, jnp.zeros((PAD_ROWS, C2), jnp.float32)], axis=0
    )
    st = _make_sc_gather()(table, idx)
    shift = st.T

    out = jnp.concatenate([former, latter, shift], axis=0)
    return out.reshape(b, 3 * C2, h, w)
